# baseline (speedup 1.0000x reference)
import jax
import jax.numpy as jnp


def kernel(mem, node_id, emb):
    B = node_id.shape[0]
    w = jnp.zeros((mem.shape[0],), jnp.int32).at[node_id].max(
        jnp.arange(1, B + 1, dtype=jnp.int32))
    rows = emb[jnp.maximum(w - 1, 0)]
    return jnp.where((w > 0)[:, None], rows, mem)


# speedup vs baseline: 1.0978x; 1.0978x over previous
import functools

import jax
import jax.numpy as jnp
from jax import lax
from jax.experimental import pallas as pl
from jax.experimental.pallas import tpu as pltpu
from jax.experimental.pallas import tpu_sc as plsc

GRAPH = 1_000_000
EMB_D = 32
BATCH = 16384

NC = 2
NS = 16
NW = NC * NS

SPAN = 31232
TAIL = GRAPH - (NW - 1) * SPAN
CHUNK = 256


def _widen_body(src_ref, dst_ref):
    x = src_ref[...]
    dst_ref[...] = jnp.pad(x, ((0, 0), (0, 128 - EMB_D)))


_emb_widen = pl.pallas_call(
    _widen_body,
    out_shape=jax.ShapeDtypeStruct((BATCH, 128), jnp.float32),
    grid=(8,),
    in_specs=[pl.BlockSpec((BATCH // 8, EMB_D), lambda i: (i, 0))],
    out_specs=pl.BlockSpec((BATCH // 8, 128), lambda i: (i, 0)),
)

NTAIL = GRAPH % 128


def _tail_body(src_ref, dst_ref):
    xt = src_ref[...].T
    dst_ref[...] = jnp.pad(xt[:NTAIL, :], ((0, 0), (0, 128 - EMB_D)))


_tail_stage = pl.pallas_call(
    _tail_body,
    out_shape=jax.ShapeDtypeStruct((NTAIL, 128), jnp.float32),
    grid=(1,),
    in_specs=[pl.BlockSpec((EMB_D, 128), lambda i: (0, GRAPH // 128))],
    out_specs=pl.BlockSpec((NTAIL, 128), lambda i: (0, 0)),
)


_mesh = plsc.VectorSubcoreMesh(core_axis_name="c", subcore_axis_name="s")


def _iota16():
    return lax.iota(jnp.int32, 16)


@functools.partial(
    pl.kernel,
    out_type=(jax.ShapeDtypeStruct((EMB_D, GRAPH), jnp.float32),
              jax.ShapeDtypeStruct((NTAIL, 128), jnp.float32)),
    mesh=_mesh,
    scratch_types=[
        pltpu.VMEM((BATCH,), jnp.int32),
        pltpu.VMEM((TAIL,), jnp.int32),
        pltpu.VMEM((EMB_D, CHUNK), jnp.float32),
        pltpu.VMEM((NTAIL, 128), jnp.float32),
        pltpu.VMEM((128,), jnp.int32),
        pltpu.VMEM((128,), jnp.int32),
        pltpu.VMEM((CHUNK,), jnp.int32),
        pltpu.VMEM((128, 128), jnp.float32),
        pltpu.VMEM((128, 128), jnp.float32),
        pltpu.SemaphoreType.DMA,
    ],
    compiler_params=pltpu.CompilerParams(needs_layout_passes=False),
)
def _sc_fused(nid_hbm, embw_hbm, tailw_hbm, memt_hbm, outt_hbm, tail_hbm,
              nid_v, claim, cbounce, tbuf, ibufa, ibufb, cbuf,
              rowsa, rowsb, gsem):
    wid = lax.axis_index("s") * NC + lax.axis_index("c")
    col0 = wid * SPAN
    span = jnp.where(wid == NW - 1, TAIL, SPAN)
    iota = _iota16()

    pltpu.sync_copy(nid_hbm, nid_v)

    def clr_body(c, _):
        claim[pl.ds(c * 16, 16)] = jnp.full((16,), -1, jnp.int32)
        return 0

    lax.fori_loop(0, TAIL // 16, clr_body, 0)

    def claim_body(c, _):
        v = nid_v[pl.ds(c * 16, 16)]
        m = (v >= col0) & (v < col0 + span)
        mcnt = jnp.sum(m.astype(jnp.int32))
        ivec = c * 16 + iota
        vl = jnp.where(m, v - col0, 0)

        @pl.when(mcnt == 1)
        def _():
            plsc.store_scatter(claim, [vl], ivec, mask=m)

        @pl.when(mcnt > 1)
        def _():
            for lane in range(16):
                plsc.store_scatter(claim, [vl], ivec,
                                   mask=m & (iota == lane))
        return 0

    lax.fori_loop(0, BATCH // 16, claim_body, 0)

    def make_apply(ccols, row_major):
        two_blocks = ccols > 128

        def apply_chunk(local0, buf):
            def scan_g(g, nd):
                cl = claim[pl.ds(local0 + g * 16, 16)]
                d = cl >= 0
                cs = plsc.cumsum(d.astype(jnp.int32))
                pos = nd + cs - 1
                da = d & (pos < 128)
                plsc.store_scatter(ibufa, [jnp.where(da, pos, 0)], cl,
                                   mask=da)
                if two_blocks:
                    db = d & (pos >= 128)
                    plsc.store_scatter(ibufb, [jnp.where(db, pos - 128, 0)],
                                       cl, mask=db)
                plsc.store_scatter(cbuf, [jnp.where(d, pos, 0)],
                                   g * 16 + iota, mask=d)
                return nd + jnp.sum(d.astype(jnp.int32))

            nd = lax.fori_loop(0, ccols // 16, scan_g, jnp.int32(0))

            @pl.when(nd > 0)
            def _():
                zeros = jnp.zeros((16,), jnp.int32)
                first = plsc.load_gather(ibufa, [zeros])

                def pad_a(g, _):
                    pos = g * 16 + iota
                    plsc.store_scatter(ibufa, [pos], first, mask=pos >= nd)
                    return 0

                lax.fori_loop(nd // 16, 8, pad_a, 0)
                pltpu.async_copy(embw_hbm.at[ibufa], rowsa, gsem).wait()

                if two_blocks:
                    @pl.when(nd > 128)
                    def _():
                        def pad_b(g, _):
                            pos = 128 + g * 16 + iota
                            plsc.store_scatter(ibufb, [pos - 128], first,
                                               mask=pos >= nd)
                            return 0

                        lax.fori_loop((nd - 128) // 16, 8, pad_b, 0)
                        pltpu.async_copy(embw_hbm.at[ibufb], rowsb,
                                         gsem).wait()

                def app_g(g, _):
                    pos = g * 16 + iota
                    lm = pos < nd
                    cloc = cbuf[pl.ds(g * 16, 16)]
                    cloc = jnp.where(lm, cloc, 0)
                    rpos = jnp.where(lm, pos, 0)
                    for f in range(EMB_D):
                        fv = jnp.full((16,), f, jnp.int32)
                        if two_blocks:
                            la = lm & (rpos < 128)
                            lb = lm & (rpos >= 128)
                            va = plsc.load_gather(
                                rowsa, [jnp.where(la, rpos, 0), fv],
                                mask=la)
                            vb = plsc.load_gather(
                                rowsb, [jnp.where(lb, rpos - 128, 0), fv],
                                mask=lb)
                            vals = jnp.where(rpos < 128, va, vb)
                        else:
                            vals = plsc.load_gather(rowsa, [rpos, fv],
                                                    mask=lm)
                        if row_major:
                            plsc.store_scatter(buf, [cloc, fv], vals,
                                               mask=lm)
                        else:
                            plsc.store_scatter(buf, [fv, cloc], vals,
                                               mask=lm)
                    return 0

                lax.fori_loop(0, (nd + 15) // 16, app_g, 0)

        return apply_chunk

    apply_full = make_apply(CHUNK, row_major=False)
    apply_tail = make_apply(NTAIL, row_major=True)

    nfull = span // CHUNK

    def chunk_body(j, _):
        local0 = j * CHUNK
        pltpu.sync_copy(memt_hbm.at[:, pl.ds(col0 + local0, CHUNK)], cbounce)
        apply_full(local0, cbounce)
        pltpu.sync_copy(cbounce, outt_hbm.at[:, pl.ds(col0 + local0, CHUNK)])
        return 0

    lax.fori_loop(0, nfull, chunk_body, 0)

    @pl.when(wid == NW - 1)
    def _():
        pltpu.sync_copy(tailw_hbm, tbuf)
        apply_tail(nfull * CHUNK, tbuf)
        pltpu.sync_copy(tbuf, tail_hbm)


def kernel(mem, node_id, emb):
    embw = _emb_widen(emb)
    tailw = _tail_stage(mem.T)
    outt, tail = _sc_fused(node_id.astype(jnp.int32), embw, tailw, mem.T)
    out = outt.T
    return lax.dynamic_update_slice(out, tail[:, :EMB_D], (GRAPH - NTAIL, 0))
